# baseline (speedup 1.0000x reference)
import functools

import jax
import jax.numpy as jnp
from jax import lax
from jax.experimental import pallas as pl
from jax.experimental.pallas import tpu as pltpu
from jax.experimental.pallas import tpu_sc as plsc

N_BIO = 1000000
N_FACTORS = 64
BATCH = 16384

_info = plsc.get_sparse_core_info()
_NC = _info.num_cores
_NS = _info.num_subcores
_NW = _NC * _NS
_BPW = BATCH // _NW
_NFLAT = _BPW * N_FACTORS
_CH = 128
_NCHUNK = _NFLAT // _CH

_mesh = plsc.VectorSubcoreMesh(core_axis_name="c", subcore_axis_name="s")


@functools.partial(
    pl.kernel,
    mesh=_mesh,
    out_type=jax.ShapeDtypeStruct((BATCH * N_FACTORS,), jnp.float32),
    scratch_types=[
        pltpu.VMEM((_BPW,), jnp.int32),
        pltpu.VMEM((_NFLAT,), jnp.int32),
        pltpu.VMEM((_NFLAT,), jnp.float32),
        pltpu.SemaphoreType.DMA,
    ],
    compiler_params=pltpu.CompilerParams(needs_layout_passes=False),
)
def _gather_kernel(idx_hbm, table_hbm, out_hbm, idx_v, flat_v, rows_v, sem):
    wid = lax.axis_index("s") * _NC + lax.axis_index("c")
    base = wid * _BPW
    lanes = lax.iota(jnp.int32, 16)
    pltpu.sync_copy(idx_hbm.at[pl.ds(base, _BPW)], idx_v)

    def expand_body(v, _):
        i_local = v >> 2
        f0 = (v & 3) * 16
        idx = plsc.load_gather(idx_v, [jnp.full((16,), 0, jnp.int32) + i_local])
        flat_v[pl.ds(v * 16, 16)] = (f0 + lanes) * N_BIO + idx
        return 0

    lax.fori_loop(0, _NFLAT // 16, expand_body, 0)

    def fire_body(c, _):
        pltpu.make_async_copy(
            table_hbm.at[flat_v.at[pl.ds(c * _CH, _CH)]],
            rows_v.at[pl.ds(c * _CH, _CH)], sem
        ).start()
        return 0

    lax.fori_loop(0, _NCHUNK, fire_body, 0)
    pltpu.make_async_copy(
        table_hbm.at[pl.ds(0, _NFLAT)], rows_v, sem
    ).wait()

    pltpu.sync_copy(rows_v, out_hbm.at[pl.ds(base * N_FACTORS, _NFLAT)])


def kernel(idxs, bio_factors):
    table_flat = bio_factors.T.reshape(-1)
    out = _gather_kernel(idxs.astype(jnp.int32), table_flat)
    return out.reshape(BATCH, N_FACTORS)


# speedup vs baseline: 8.0444x; 8.0444x over previous
import functools

import jax
import jax.numpy as jnp
from jax import lax
from jax.experimental import pallas as pl
from jax.experimental.pallas import tpu as pltpu
from jax.experimental.pallas import tpu_sc as plsc

N_BIO = 1000000
N_FACTORS = 64
BATCH = 16384

_info = plsc.get_sparse_core_info()
_NC = _info.num_cores
_NS = _info.num_subcores
_NW = _NC * _NS
_BPW = BATCH // _NW
_CH = 128
_NCH = _BPW // _CH

_mesh = plsc.VectorSubcoreMesh(core_axis_name="c", subcore_axis_name="s")


@functools.partial(
    pl.kernel,
    mesh=_mesh,
    out_type=jax.ShapeDtypeStruct((_NW, _NCH, _CH, N_FACTORS), jnp.float32),
    scratch_types=[
        pltpu.VMEM((_NCH, _CH), jnp.int32),
        pltpu.VMEM((_NCH, _CH, N_FACTORS), jnp.float32),
        pltpu.SemaphoreType.DMA,
    ],
    compiler_params=pltpu.CompilerParams(use_tc_tiling_on_sc=False),
)
def _gather_kernel(idx_hbm, table_hbm, out_hbm, idx_v, rows_v, sem):
    wid = lax.axis_index("s") * _NC + lax.axis_index("c")
    pltpu.sync_copy(idx_hbm.at[wid], idx_v)
    copies = [
        pltpu.async_copy(table_hbm.at[idx_v.at[j]], rows_v.at[j], sem)
        for j in range(_NCH)
    ]
    for c in copies:
        c.wait()
    pltpu.sync_copy(rows_v, out_hbm.at[wid])


def kernel(idxs, bio_factors):
    idx3 = idxs.astype(jnp.int32).reshape(_NW, _NCH, _CH)
    out = _gather_kernel(idx3, bio_factors)
    return out.reshape(BATCH, N_FACTORS)
